# baseline (speedup 1.0000x reference)
import functools

import jax
import jax.numpy as jnp
from jax import lax
from jax.experimental import pallas as pl
from jax.experimental.pallas import tpu as pltpu
from jax.experimental.pallas import tpu_sc as plsc

N = 10000
E = 320000
D = 128
H = 128
DE = 16
OUT = 64
B = 8192

NC = 2
NS = 16
NW = NC * NS
CHUNK = 128
NCH = 2560
CH_W = NCH // NW
GRP = 8
NGRP = CH_W // GRP
EPAD = NCH * CHUNK
EC = 128
NCHE = EPAD // EC
EC_W = NCHE // NW
EG = 16
NPAD = 10240
RPT = NPAD // NS
NZB = RPT // CHUNK


def _dot(a, b):
    return lax.dot_general(a, b, (((1,), (0,)), ((), ())),
                           preferred_element_type=jnp.float32)


_sc_mesh = plsc.VectorSubcoreMesh(core_axis_name="c", subcore_axis_name="s")


CS_COL = 17


@functools.partial(
    pl.kernel,
    out_type=jax.ShapeDtypeStruct((NC * NPAD, H), jnp.float32),
    mesh=_sc_mesh,
    scratch_types=[
        pltpu.VMEM((GRP, CHUNK), jnp.int32),
        pltpu.VMEM((GRP, CHUNK), jnp.int32),
        pltpu.VMEM((CHUNK, H), jnp.float32),
        pltpu.VMEM((CHUNK, H), jnp.float32),
        pltpu.VMEM_SHARED((NPAD, H), jnp.float32),
        pltpu.SemaphoreType.DMA,
    ],
)
def _sc_stats(src_hbm, dst_hbm, val_hbm, ones_hbm, zeros_hbm, out_hbm,
              src_v, dst_v, ones_v, val0, acc, sem0):
    c = lax.axis_index("c")
    s = lax.axis_index("s")
    base = (c * NS + s) * CH_W
    pltpu.sync_copy(zeros_hbm, val0)

    def zblk(j, carry):
        pltpu.sync_copy(val0, acc.at[pl.ds(s * RPT + j * CHUNK, CHUNK)])
        return carry

    lax.fori_loop(0, NZB, zblk, 0)
    pltpu.sync_copy(ones_hbm, ones_v)
    plsc.subcore_barrier()

    def group(g, carry):
        gb = base + g * GRP
        pltpu.sync_copy(src_hbm.at[pl.ds(gb, GRP)], src_v)
        pltpu.sync_copy(dst_hbm.at[pl.ds(gb, GRP)], dst_v)

        def step(t, carry2):
            pltpu.sync_copy(val_hbm.at[pl.ds((gb + t) * CHUNK, CHUNK)], val0)
            pltpu.sync_copy(val0, acc.at[dst_v.at[t]], add=True)
            pltpu.sync_copy(ones_v, acc.at[src_v.at[t]], add=True)
            return carry2

        lax.fori_loop(0, GRP, step, 0)
        return carry

    lax.fori_loop(0, NGRP, group, 0)
    plsc.subcore_barrier()
    ob = c * NPAD + s * RPT

    def wblk(j, carry):
        pltpu.sync_copy(acc.at[pl.ds(s * RPT + j * CHUNK, CHUNK)], val0)
        pltpu.sync_copy(val0, out_hbm.at[pl.ds(ob + j * CHUNK, CHUNK)])
        return carry

    lax.fori_loop(0, NZB, wblk, 0)


@functools.partial(
    pl.kernel,
    out_type=jax.ShapeDtypeStruct((NC * NPAD, H), jnp.float32),
    mesh=_sc_mesh,
    scratch_types=[
        pltpu.VMEM((EG, EC), jnp.int32),
        pltpu.VMEM((EG, EC), jnp.int32),
        pltpu.VMEM((EC, H), jnp.float32),
        pltpu.VMEM((EC, H), jnp.float32),
        pltpu.VMEM_SHARED((NPAD, H), jnp.float32),
        pltpu.SemaphoreType.DMA,
        pltpu.SemaphoreType.DMA,
    ],
)
def _sc_spmm(y_hbm, src_hbm, dst_hbm, zeros_hbm, out_hbm,
             src_v, dst_v, buf0, buf1, acc, sem0, sem1):
    c = lax.axis_index("c")
    s = lax.axis_index("s")
    base = (c * NS + s) * EC_W
    bufs = (buf0, buf1)
    sems = (sem0, sem1)
    pltpu.sync_copy(zeros_hbm, buf0)

    def zblk(j, carry):
        pltpu.sync_copy(buf0, acc.at[pl.ds(s * RPT + j * EC, EC)])
        return carry

    lax.fori_loop(0, RPT // EC, zblk, 0)
    plsc.subcore_barrier()

    def group(g, carry):
        gb = base + g * EG
        pltpu.sync_copy(src_hbm.at[pl.ds(gb, EG)], src_v)
        pltpu.sync_copy(dst_hbm.at[pl.ds(gb, EG)], dst_v)
        pltpu.async_copy(y_hbm.at[src_v.at[0]], buf0, sem0)

        def pair(q, carry2):
            for k in range(2):
                t = 2 * q + k
                pltpu.make_async_copy(y_hbm.at[src_v.at[t]], bufs[k],
                                      sems[k]).wait()

                @pl.when(t + 1 < EG)
                def _():
                    pltpu.async_copy(y_hbm.at[src_v.at[t + 1]], bufs[1 - k],
                                     sems[1 - k])

                pltpu.sync_copy(bufs[k], acc.at[dst_v.at[t]], add=True)
            return carry2

        lax.fori_loop(0, EG // 2, pair, 0)
        return carry

    lax.fori_loop(0, EC_W // EG, group, 0)
    plsc.subcore_barrier()
    ob = c * NPAD + s * RPT

    def oblk(j, carry):
        pltpu.sync_copy(acc.at[pl.ds(s * RPT + j * EC, EC)], buf0)
        pltpu.sync_copy(buf0, out_hbm.at[pl.ds(ob + j * EC, EC)])
        return carry

    lax.fori_loop(0, RPT // EC, oblk, 0)


LCH = B // CHUNK
LCH_W = LCH // NW


@functools.partial(
    pl.kernel,
    out_type=(jax.ShapeDtypeStruct((B, H), jnp.float32),
              jax.ShapeDtypeStruct((B, H), jnp.float32)),
    mesh=_sc_mesh,
    scratch_types=[
        pltpu.VMEM((LCH_W, CHUNK), jnp.int32),
        pltpu.VMEM((LCH_W, CHUNK), jnp.int32),
        pltpu.VMEM((CHUNK, H), jnp.float32),
        pltpu.VMEM((CHUNK, H), jnp.float32),
        pltpu.SemaphoreType.DMA,
        pltpu.SemaphoreType.DMA,
    ],
)
def _sc_labgather(xl_hbm, xr_hbm, li_hbm, ri_hbm, ol_hbm, or_hbm,
                  li_v, ri_v, buf0, buf1, sem0, sem1):
    c = lax.axis_index("c")
    s = lax.axis_index("s")
    base = (c * NS + s) * LCH_W
    pltpu.sync_copy(li_hbm.at[pl.ds(base, LCH_W)], li_v)
    pltpu.sync_copy(ri_hbm.at[pl.ds(base, LCH_W)], ri_v)

    def body(t, carry):
        o = (base + t) * CHUNK
        pltpu.async_copy(xl_hbm.at[li_v.at[t]], buf0, sem0)
        pltpu.async_copy(xr_hbm.at[ri_v.at[t]], buf1, sem1)
        pltpu.make_async_copy(xl_hbm.at[li_v.at[t]], buf0, sem0).wait()
        pltpu.sync_copy(buf0, ol_hbm.at[pl.ds(o, CHUNK)])
        pltpu.make_async_copy(xr_hbm.at[ri_v.at[t]], buf1, sem1).wait()
        pltpu.sync_copy(buf1, or_hbm.at[pl.ds(o, CHUNK)])
        return carry

    lax.fori_loop(0, LCH_W, body, 0)


def _tca_body(x_ref, wn_ref, bn_ref, ws_ref, bs_ref, wa_ref, ba_ref,
              cs1_ref, y_ref, self_ref):
    x = x_ref[...]
    xn = _dot(x, wn_ref[...]) + bn_ref[...]
    s = _dot(xn, wa_ref[...]) + ba_ref[...]
    s = jnp.where(s >= 0.0, s, 0.2 * s)
    m = jnp.max(s)
    e = jnp.exp(s - m)
    z = jnp.sum(cs1_ref[...] * e)
    y_ref[...] = xn * (e / z)
    self_ref[...] = _dot(x, ws_ref[...]) + bs_ref[...]


def _tca(x, p, cs1):
    return pl.pallas_call(
        _tca_body,
        out_shape=(jax.ShapeDtypeStruct((N, H), jnp.float32),
                   jax.ShapeDtypeStruct((N, H), jnp.float32)),
    )(x, p["Wneigh"], p["bneigh"].reshape(1, H), p["Wself"],
      p["bself"].reshape(1, H), p["Watt"], p["batt"].reshape(1, 1), cs1)


def _tcb_body(relu, self_ref, y_ref, p_ref, dea_ref, cd_ref, we_ref, be_ref,
              bias_ref, o_ref):
    p = p_ref[:N, :] + p_ref[NPAD:NPAD + N, :]
    we = we_ref[...]
    be = jnp.sum(be_ref[...])
    ef_loop = jnp.sum(we) + be
    ef = jnp.sum(dea_ref[...] * we, axis=1, keepdims=True) + cd_ref[...] * be + ef_loop
    h = self_ref[...] + y_ref[...] + p + ef * (1.0 / 20.0) + bias_ref[...]
    o_ref[...] = jnp.maximum(h, 0.0) if relu else h


def _tcb(self_feat, y, p_part, dea, cd, p, relu):
    return pl.pallas_call(
        functools.partial(_tcb_body, relu),
        out_shape=jax.ShapeDtypeStruct((N, H), jnp.float32),
    )(self_feat, y, p_part, dea, cd, p["Wedge"].reshape(1, DE),
      p["bedge"].reshape(1, 1), p["bias"].reshape(1, H))


def _head_body(gl_ref, gr_ref, w1_ref, b1_ref, w2_ref, b2_ref, o_ref):
    h = _dot(gl_ref[...], w1_ref[:H, :]) + _dot(gr_ref[...], w1_ref[H:, :])
    h = jnp.maximum(h + b1_ref[...], 0.0)
    o_ref[...] = _dot(h, w2_ref[...]) + b2_ref[...]


def _head(gl, gr, params):
    W1, b1 = params["fc1"]
    W2, b2 = params["fc2"]
    return pl.pallas_call(
        _head_body,
        out_shape=jax.ShapeDtypeStruct((B, OUT), jnp.float32),
    )(gl, gr, W1, b1.reshape(1, H), W2, b2.reshape(1, OUT))


def _prep_edges(ei, ea):
    src = ei[0].astype(jnp.int32)
    dst = ei[1].astype(jnp.int32)
    pad = EPAD - E
    src_g = jnp.pad(src, (0, pad))
    src_s = jnp.pad(src, (0, pad), constant_values=N)
    dst_p = jnp.pad(dst, (0, pad), constant_values=N)
    val = jnp.concatenate(
        [jnp.ones((E, 1), jnp.float32), ea,
         jnp.zeros((E, H - DE - 1), jnp.float32)], axis=1)
    val_p = jnp.pad(val, ((0, pad), (0, 0)))
    return src_g, src_s, dst_p, val_p


def kernel(x_l, edge_index_l, edge_attr_l, x_r, edge_index_r, edge_attr_r,
           labels, params):
    zeros128 = jnp.zeros((CHUNK, H), jnp.float32)
    zeros64 = jnp.zeros((EC, H), jnp.float32)
    onescs = jnp.zeros((CHUNK, H), jnp.float32).at[:, CS_COL].set(1.0)

    hs = {}
    for side, x, ei, ea in (("l", x_l, edge_index_l, edge_attr_l),
                            ("r", x_r, edge_index_r, edge_attr_r)):
        src_g, src_s, dst_p, val_p = _prep_edges(ei, ea)
        stats = _sc_stats(src_s.reshape(NCH, CHUNK), dst_p.reshape(NCH, CHUNK),
                          val_p, onescs, zeros128)
        stats = stats.reshape(NC, NPAD, H)
        st = stats[0, :N] + stats[1, :N]
        cd = st[:, 0:1]
        dea = st[:, 1:DE + 1]
        cs1 = st[:, CS_COL:CS_COL + 1] + 1.0
        h = x
        for layer in ("1", "2"):
            p = params["c" + layer + side]
            y, self_feat = _tca(h, p, cs1)
            p_part = _sc_spmm(y, src_g.reshape(NCHE, EC),
                              dst_p.reshape(NCHE, EC), zeros64)
            h = _tcb(self_feat, y, p_part, dea, cd, p, relu=(layer == "1"))
        hs[side] = h

    li = labels[:, 0].astype(jnp.int32).reshape(LCH, CHUNK)
    ri = labels[:, 1].astype(jnp.int32).reshape(LCH, CHUNK)
    gl, gr = _sc_labgather(hs["l"], hs["r"], li, ri)
    return _head(gl, gr, params)


# speedup vs baseline: 1.1610x; 1.1610x over previous
import functools

import jax
import jax.numpy as jnp
from jax import lax
from jax.experimental import pallas as pl
from jax.experimental.pallas import tpu as pltpu
from jax.experimental.pallas import tpu_sc as plsc

N = 10000
E = 320000
D = 128
H = 128
DE = 16
OUT = 64
B = 8192

NC = 2
NS = 16
NW = NC * NS
CHUNK = 128
NCH = 2560
CH_W = NCH // NW
GRP = 8
NGRP = CH_W // GRP
EPAD = NCH * CHUNK
EC = 64
NCHE = EPAD // EC
EC_W = NCHE // NW
EG = 16
NPAD = 10240
RPT = NPAD // NS
NZB = RPT // CHUNK


def _dot(a, b):
    return lax.dot_general(a, b, (((1,), (0,)), ((), ())),
                           preferred_element_type=jnp.float32)


_sc_mesh = plsc.VectorSubcoreMesh(core_axis_name="c", subcore_axis_name="s")


CS_COL = 17


@functools.partial(
    pl.kernel,
    out_type=jax.ShapeDtypeStruct((NC * NPAD, H), jnp.float32),
    mesh=_sc_mesh,
    scratch_types=[
        pltpu.VMEM((GRP, CHUNK), jnp.int32),
        pltpu.VMEM((GRP, CHUNK), jnp.int32),
        pltpu.VMEM((CHUNK, H), jnp.float32),
        pltpu.VMEM((CHUNK, H), jnp.float32),
        pltpu.VMEM_SHARED((NPAD, H), jnp.float32),
        pltpu.SemaphoreType.DMA,
    ],
)
def _sc_stats(src_hbm, dst_hbm, val_hbm, ones_hbm, zeros_hbm, out_hbm,
              src_v, dst_v, ones_v, val0, acc, sem0):
    c = lax.axis_index("c")
    s = lax.axis_index("s")
    base = (c * NS + s) * CH_W
    pltpu.sync_copy(zeros_hbm, val0)

    def zblk(j, carry):
        pltpu.sync_copy(val0, acc.at[pl.ds(s * RPT + j * CHUNK, CHUNK)])
        return carry

    lax.fori_loop(0, NZB, zblk, 0)
    pltpu.sync_copy(ones_hbm, ones_v)
    plsc.subcore_barrier()

    def group(g, carry):
        gb = base + g * GRP
        pltpu.sync_copy(src_hbm.at[pl.ds(gb, GRP)], src_v)
        pltpu.sync_copy(dst_hbm.at[pl.ds(gb, GRP)], dst_v)

        def step(t, carry2):
            pltpu.sync_copy(val_hbm.at[pl.ds((gb + t) * CHUNK, CHUNK)], val0)
            pltpu.sync_copy(val0, acc.at[dst_v.at[t]], add=True)
            pltpu.sync_copy(ones_v, acc.at[src_v.at[t]], add=True)
            return carry2

        lax.fori_loop(0, GRP, step, 0)
        return carry

    lax.fori_loop(0, NGRP, group, 0)
    plsc.subcore_barrier()
    ob = c * NPAD + s * RPT

    def wblk(j, carry):
        pltpu.sync_copy(acc.at[pl.ds(s * RPT + j * CHUNK, CHUNK)], val0)
        pltpu.sync_copy(val0, out_hbm.at[pl.ds(ob + j * CHUNK, CHUNK)])
        return carry

    lax.fori_loop(0, NZB, wblk, 0)


@functools.partial(
    pl.kernel,
    out_type=jax.ShapeDtypeStruct((NC * NPAD, H), jnp.float32),
    mesh=_sc_mesh,
    scratch_types=[
        pltpu.VMEM((EG, EC), jnp.int32),
        pltpu.VMEM((EG, EC), jnp.int32),
        pltpu.VMEM((EC, H), jnp.float32),
        pltpu.VMEM((EC, H), jnp.float32),
        pltpu.VMEM((EC, H), jnp.float32),
        pltpu.VMEM((EC, H), jnp.float32),
        pltpu.VMEM((EC, H), jnp.float32),
        pltpu.VMEM_SHARED((NPAD, H), jnp.float32),
        pltpu.SemaphoreType.DMA,
        pltpu.SemaphoreType.DMA,
        pltpu.SemaphoreType.DMA,
        pltpu.SemaphoreType.DMA,
        pltpu.SemaphoreType.DMA,
    ],
)
def _sc_spmm(y_hbm, src_hbm, dst_hbm, zeros_hbm, out_hbm,
             src_v, dst_v, buf0, buf1, buf2, buf3, buf4, acc,
             sem0, sem1, sem2, sem3, sem4):
    c = lax.axis_index("c")
    s = lax.axis_index("s")
    base = (c * NS + s) * EC_W
    bufs = (buf0, buf1, buf2, buf3, buf4)
    sems = (sem0, sem1, sem2, sem3, sem4)
    pltpu.sync_copy(zeros_hbm, buf0)

    def zblk(j, carry):
        pltpu.sync_copy(buf0, acc.at[pl.ds(s * RPT + j * EC, EC)])
        return carry

    lax.fori_loop(0, RPT // EC, zblk, 0)
    plsc.subcore_barrier()

    def group(g, carry):
        gb = base + g * EG
        pltpu.sync_copy(src_hbm.at[pl.ds(gb, EG)], src_v)
        pltpu.sync_copy(dst_hbm.at[pl.ds(gb, EG)], dst_v)
        for k in range(4):
            pltpu.async_copy(y_hbm.at[src_v.at[k]], bufs[k], sems[k])

        def quint(q, carry2):
            for k in range(5):
                t = 5 * q + k
                pltpu.make_async_copy(y_hbm.at[src_v.at[t]], bufs[k],
                                      sems[k]).wait()
                kn = (k + 4) % 5

                @pl.when(t + 4 < EG)
                def _():
                    pltpu.async_copy(y_hbm.at[src_v.at[t + 4]], bufs[kn],
                                     sems[kn])

                pltpu.sync_copy(bufs[k], acc.at[dst_v.at[t]], add=True)
            return carry2

        lax.fori_loop(0, (EG - 1) // 5, quint, 0)
        t = EG - 1
        k = t % 5
        pltpu.make_async_copy(y_hbm.at[src_v.at[t]], bufs[k], sems[k]).wait()
        pltpu.sync_copy(bufs[k], acc.at[dst_v.at[t]], add=True)
        return carry

    lax.fori_loop(0, EC_W // EG, group, 0)
    plsc.subcore_barrier()
    ob = c * NPAD + s * RPT

    def oblk(j, carry):
        pltpu.sync_copy(acc.at[pl.ds(s * RPT + j * EC, EC)], buf0)
        pltpu.sync_copy(buf0, out_hbm.at[pl.ds(ob + j * EC, EC)])
        return carry

    lax.fori_loop(0, RPT // EC, oblk, 0)


LCH = B // CHUNK
LCH_W = LCH // NW


@functools.partial(
    pl.kernel,
    out_type=(jax.ShapeDtypeStruct((B, H), jnp.float32),
              jax.ShapeDtypeStruct((B, H), jnp.float32)),
    mesh=_sc_mesh,
    scratch_types=[
        pltpu.VMEM((LCH_W, CHUNK), jnp.int32),
        pltpu.VMEM((LCH_W, CHUNK), jnp.int32),
        pltpu.VMEM((CHUNK, H), jnp.float32),
        pltpu.VMEM((CHUNK, H), jnp.float32),
        pltpu.SemaphoreType.DMA,
        pltpu.SemaphoreType.DMA,
    ],
)
def _sc_labgather(xl_hbm, xr_hbm, li_hbm, ri_hbm, ol_hbm, or_hbm,
                  li_v, ri_v, buf0, buf1, sem0, sem1):
    c = lax.axis_index("c")
    s = lax.axis_index("s")
    base = (c * NS + s) * LCH_W
    pltpu.sync_copy(li_hbm.at[pl.ds(base, LCH_W)], li_v)
    pltpu.sync_copy(ri_hbm.at[pl.ds(base, LCH_W)], ri_v)

    def body(t, carry):
        o = (base + t) * CHUNK
        pltpu.async_copy(xl_hbm.at[li_v.at[t]], buf0, sem0)
        pltpu.async_copy(xr_hbm.at[ri_v.at[t]], buf1, sem1)
        pltpu.make_async_copy(xl_hbm.at[li_v.at[t]], buf0, sem0).wait()
        pltpu.sync_copy(buf0, ol_hbm.at[pl.ds(o, CHUNK)])
        pltpu.make_async_copy(xr_hbm.at[ri_v.at[t]], buf1, sem1).wait()
        pltpu.sync_copy(buf1, or_hbm.at[pl.ds(o, CHUNK)])
        return carry

    lax.fori_loop(0, LCH_W, body, 0)


def _tca_body(x_ref, wn_ref, bn_ref, ws_ref, bs_ref, wa_ref, ba_ref,
              cs1_ref, y_ref, self_ref):
    x = x_ref[...]
    xn = _dot(x, wn_ref[...]) + bn_ref[...]
    s = _dot(xn, wa_ref[...]) + ba_ref[...]
    s = jnp.where(s >= 0.0, s, 0.2 * s)
    m = jnp.max(s)
    e = jnp.exp(s - m)
    z = jnp.sum(cs1_ref[...] * e)
    y_ref[...] = xn * (e / z)
    self_ref[...] = _dot(x, ws_ref[...]) + bs_ref[...]


def _tca(x, p, cs1):
    return pl.pallas_call(
        _tca_body,
        out_shape=(jax.ShapeDtypeStruct((N, H), jnp.float32),
                   jax.ShapeDtypeStruct((N, H), jnp.float32)),
    )(x, p["Wneigh"], p["bneigh"].reshape(1, H), p["Wself"],
      p["bself"].reshape(1, H), p["Watt"], p["batt"].reshape(1, 1), cs1)


def _tcb_body(relu, self_ref, y_ref, p_ref, dea_ref, cd_ref, we_ref, be_ref,
              bias_ref, o_ref):
    p = p_ref[:N, :] + p_ref[NPAD:NPAD + N, :]
    we = we_ref[...]
    be = jnp.sum(be_ref[...])
    ef_loop = jnp.sum(we) + be
    ef = jnp.sum(dea_ref[...] * we, axis=1, keepdims=True) + cd_ref[...] * be + ef_loop
    h = self_ref[...] + y_ref[...] + p + ef * (1.0 / 20.0) + bias_ref[...]
    o_ref[...] = jnp.maximum(h, 0.0) if relu else h


def _tcb(self_feat, y, p_part, dea, cd, p, relu):
    return pl.pallas_call(
        functools.partial(_tcb_body, relu),
        out_shape=jax.ShapeDtypeStruct((N, H), jnp.float32),
    )(self_feat, y, p_part, dea, cd, p["Wedge"].reshape(1, DE),
      p["bedge"].reshape(1, 1), p["bias"].reshape(1, H))


def _head_body(gl_ref, gr_ref, w1_ref, b1_ref, w2_ref, b2_ref, o_ref):
    h = _dot(gl_ref[...], w1_ref[:H, :]) + _dot(gr_ref[...], w1_ref[H:, :])
    h = jnp.maximum(h + b1_ref[...], 0.0)
    o_ref[...] = _dot(h, w2_ref[...]) + b2_ref[...]


def _head(gl, gr, params):
    W1, b1 = params["fc1"]
    W2, b2 = params["fc2"]
    return pl.pallas_call(
        _head_body,
        out_shape=jax.ShapeDtypeStruct((B, OUT), jnp.float32),
    )(gl, gr, W1, b1.reshape(1, H), W2, b2.reshape(1, OUT))


def _prep_edges(ei, ea):
    src = ei[0].astype(jnp.int32)
    dst = ei[1].astype(jnp.int32)
    pad = EPAD - E
    src_g = jnp.pad(src, (0, pad))
    src_s = jnp.pad(src, (0, pad), constant_values=N)
    dst_p = jnp.pad(dst, (0, pad), constant_values=N)
    val = jnp.concatenate(
        [jnp.ones((E, 1), jnp.float32), ea,
         jnp.zeros((E, H - DE - 1), jnp.float32)], axis=1)
    val_p = jnp.pad(val, ((0, pad), (0, 0)))
    return src_g, src_s, dst_p, val_p


def kernel(x_l, edge_index_l, edge_attr_l, x_r, edge_index_r, edge_attr_r,
           labels, params):
    zeros128 = jnp.zeros((CHUNK, H), jnp.float32)
    zeros64 = jnp.zeros((EC, H), jnp.float32)
    onescs = jnp.zeros((CHUNK, H), jnp.float32).at[:, CS_COL].set(1.0)

    hs = {}
    for side, x, ei, ea in (("l", x_l, edge_index_l, edge_attr_l),
                            ("r", x_r, edge_index_r, edge_attr_r)):
        src_g, src_s, dst_p, val_p = _prep_edges(ei, ea)
        stats = _sc_stats(src_s.reshape(NCH, CHUNK), dst_p.reshape(NCH, CHUNK),
                          val_p, onescs, zeros128)
        stats = stats.reshape(NC, NPAD, H)
        st = stats[0, :N] + stats[1, :N]
        cd = st[:, 0:1]
        dea = st[:, 1:DE + 1]
        cs1 = st[:, CS_COL:CS_COL + 1] + 1.0
        h = x
        for layer in ("1", "2"):
            p = params["c" + layer + side]
            y, self_feat = _tca(h, p, cs1)
            p_part = _sc_spmm(y, src_g.reshape(NCHE, EC),
                              dst_p.reshape(NCHE, EC), zeros64)
            h = _tcb(self_feat, y, p_part, dea, cd, p, relu=(layer == "1"))
        hs[side] = h

    li = labels[:, 0].astype(jnp.int32).reshape(LCH, CHUNK)
    ri = labels[:, 1].astype(jnp.int32).reshape(LCH, CHUNK)
    gl, gr = _sc_labgather(hs["l"], hs["r"], li, ri)
    return _head(gl, gr, params)
